# speedup vs baseline: 2.8107x; 2.8107x over previous
import functools

import jax
import jax.numpy as jnp
from jax import lax
from jax.experimental import pallas as pl
from jax.experimental.pallas import tpu as pltpu
from jax.experimental.pallas import tpu_sc as plsc

N_VOCAB = 100000
D_MODEL = 128
B_ROWS = 16384 * 50
NUM_WORKERS = 32
ROWS_PER_WORKER = B_ROWS // NUM_WORKERS
CHUNK = 128
NUM_CHUNKS = ROWS_PER_WORKER // CHUNK


def _gather_kernel(idx_hbm, table_hbm, out_hbm, idx_v, rows_v, sem):
    wid = lax.axis_index("s") * 2 + lax.axis_index("c")
    base = wid * ROWS_PER_WORKER

    def body(i, carry):
        off = base + i * CHUNK
        pltpu.sync_copy(idx_hbm.at[pl.ds(off, CHUNK)], idx_v)
        pltpu.async_copy(table_hbm.at[idx_v], rows_v, sem).wait()
        pltpu.sync_copy(rows_v, out_hbm.at[pl.ds(off, CHUNK)])
        return carry

    lax.fori_loop(0, NUM_CHUNKS, body, 0)


def kernel(inputs, embedding_weight):
    idx = inputs.reshape(-1).astype(jnp.int32)
    mesh = plsc.VectorSubcoreMesh(core_axis_name="c", subcore_axis_name="s")
    run = functools.partial(
        pl.kernel,
        mesh=mesh,
        out_type=jax.ShapeDtypeStruct((B_ROWS, D_MODEL), jnp.float32),
        scratch_types=[
            pltpu.VMEM((CHUNK,), jnp.int32),
            pltpu.VMEM((CHUNK, D_MODEL), jnp.float32),
            pltpu.SemaphoreType.DMA,
        ],
    )(_gather_kernel)
    out = run(idx, embedding_weight)
    return out.reshape(inputs.shape[0], inputs.shape[1], D_MODEL)


# speedup vs baseline: 3.4516x; 1.2280x over previous
import functools

import jax
import jax.numpy as jnp
from jax import lax
from jax.experimental import pallas as pl
from jax.experimental.pallas import tpu as pltpu
from jax.experimental.pallas import tpu_sc as plsc

N_VOCAB = 100000
D_MODEL = 128
B_ROWS = 16384 * 50
NUM_WORKERS = 32
ROWS_PER_WORKER = B_ROWS // NUM_WORKERS
G = 128
K = 2
CH = G * K
NUM_CHUNKS = ROWS_PER_WORKER // CH
IDX_ROWS = ROWS_PER_WORKER // G


def _gather_kernel(idx_hbm, table_hbm, out_hbm,
                   idx_v, buf0, buf1, sem0, sem1):
    wid = lax.axis_index("s") * 2 + lax.axis_index("c")
    base = wid * ROWS_PER_WORKER
    idx_base = wid * IDX_ROWS
    pltpu.sync_copy(idx_hbm.at[pl.ds(idx_base, IDX_ROWS)], idx_v)

    bufs = (buf0, buf1)
    sems = (sem0, sem1)

    def fire(chunk, buf, sem):
        for j in range(K):
            pltpu.async_copy(table_hbm.at[idx_v.at[chunk * K + j]],
                             buf.at[pl.ds(j * G, G)], sem)

    def drain(chunk, buf, sem):
        for j in range(K):
            pltpu.make_async_copy(table_hbm.at[idx_v.at[chunk * K + j]],
                                  buf.at[pl.ds(j * G, G)], sem).wait()

    fire(0, buf0, sem0)
    fire(1, buf1, sem1)

    def body(o, carry):
        for b in range(2):
            i = 2 * o + b
            buf, sem = bufs[b], sems[b]
            drain(i, buf, sem)
            pltpu.sync_copy(buf, out_hbm.at[pl.ds(base + i * CH, CH)])
            nxt = i + 2

            @pl.when(nxt < NUM_CHUNKS)
            def _():
                fire(nxt, buf, sem)
        return carry

    lax.fori_loop(0, NUM_CHUNKS // 2, body, 0)


def kernel(inputs, embedding_weight):
    idx = inputs.reshape(B_ROWS // G, G).astype(jnp.int32)
    mesh = plsc.VectorSubcoreMesh(core_axis_name="c", subcore_axis_name="s")
    run = functools.partial(
        pl.kernel,
        mesh=mesh,
        out_type=jax.ShapeDtypeStruct((B_ROWS, D_MODEL), jnp.float32),
        scratch_types=[
            pltpu.VMEM((IDX_ROWS, G), jnp.int32),
            pltpu.VMEM((CH, D_MODEL), jnp.float32),
            pltpu.VMEM((CH, D_MODEL), jnp.float32),
            pltpu.SemaphoreType.DMA,
            pltpu.SemaphoreType.DMA,
        ],
    )(_gather_kernel)
    out = run(idx, embedding_weight)
    return out.reshape(inputs.shape[0], inputs.shape[1], D_MODEL)
